# baseline (speedup 1.0000x reference)
import functools

import jax
import jax.numpy as jnp
from jax import lax
from jax.experimental import pallas as pl
from jax.experimental.pallas import tpu as pltpu
from jax.experimental.pallas import tpu_sc as plsc

N_NODES = 10000
N_PAD = 10240
N_EDGES = 320000
D = 128
K = 125
G = 40
NB = 2
ZR = 8


def _sc_segment_partials(x, ei5d):
    info = plsc.get_sparse_core_info()
    nc, ns = info.num_cores, info.num_subcores
    nw = nc * ns
    e_per_w = N_EDGES // nw
    chunks = e_per_w // K
    nsuper = chunks // G
    rows_per_tile = N_PAD // ns

    mesh = plsc.VectorSubcoreMesh(core_axis_name="c", subcore_axis_name="s")

    @functools.partial(
        pl.kernel,
        out_type=jax.ShapeDtypeStruct((nc, N_PAD, D), jnp.float32),
        mesh=mesh,
        scratch_types=[
            pltpu.VMEM((G, K), jnp.int32),
            pltpu.VMEM((G, K), jnp.int32),
            pltpu.VMEM((G, K), jnp.int32),
            [pltpu.VMEM((K, D), jnp.float32)] * NB,
            pltpu.VMEM((ZR, D), jnp.float32),
            pltpu.VMEM_SHARED((N_PAD, D), jnp.float32),
            [pltpu.SemaphoreType.DMA] * NB,
            [pltpu.SemaphoreType.DMA] * NB,
            pltpu.SemaphoreType.DMA,
        ],
    )
    def seg_kernel(x_hbm, ei_hbm, out_hbm,
                   sidx, sidx2, didx, rb, zbuf, acc, gsem, ssem, isem):
        c = lax.axis_index("c")
        s = lax.axis_index("s")
        wid = s * nc + c

        pltpu.sync_copy(ei_hbm.at[0, wid, 0], sidx)
        pltpu.sync_copy(ei_hbm.at[1, wid, 0], didx)
        for b in range(NB):
            pltpu.async_copy(x_hbm.at[sidx.at[b]], rb[b], gsem[b])
        pltpu.async_copy(ei_hbm.at[0, wid, 1], sidx2, isem)

        for r in range(ZR):
            for j in range(D // 16):
                zbuf[r, pl.ds(16 * j, 16)] = jnp.zeros((16,), jnp.float32)
        row_base = s * rows_per_tile

        @pl.loop(0, rows_per_tile // ZR)
        def _zero(k):
            pltpu.sync_copy(zbuf, acc.at[pl.ds(row_base + ZR * k, ZR)])

        plsc.subcore_barrier()

        for si in range(nsuper):
            six = sidx if si == 0 else sidx2
            if si > 0:
                pltpu.make_async_copy(ei_hbm.at[0, wid, 1], sidx2, isem).wait()
                for b in range(NB):
                    pltpu.async_copy(x_hbm.at[six.at[b]], rb[b], gsem[b])
                pltpu.sync_copy(ei_hbm.at[1, wid, si], didx)

            @pl.loop(0, G, step=NB)
            def _edges(g):
                for b in range(NB):
                    pltpu.make_async_copy(
                        x_hbm.at[six.at[g + b]], rb[b], gsem[b]).wait()
                    pltpu.sync_copy(rb[b], acc.at[didx.at[g + b]], add=True)

                    @pl.when(g + NB + b < G)
                    def _(b=b):
                        pltpu.async_copy(
                            x_hbm.at[six.at[g + NB + b]], rb[b], gsem[b])

        plsc.subcore_barrier()
        pltpu.sync_copy(acc.at[pl.ds(row_base, rows_per_tile)],
                        out_hbm.at[c, pl.ds(row_base, rows_per_tile)])

    return seg_kernel(x, ei5d)


def _tc_apply(partials, W, b):
    nc = partials.shape[0]
    BM = 5000

    def mm(p_ref, w_ref, b_ref, o_ref):
        agg = p_ref[0]
        for i in range(1, nc):
            agg = agg + p_ref[i]
        o_ref[...] = (
            jnp.dot(agg, w_ref[...], preferred_element_type=jnp.float32)
            + b_ref[...]
        )

    return pl.pallas_call(
        mm,
        grid=(N_NODES // BM,),
        in_specs=[
            pl.BlockSpec((nc, BM, D), lambda i: (0, i, 0)),
            pl.BlockSpec((D, D), lambda i: (0, 0)),
            pl.BlockSpec((1, D), lambda i: (0, 0)),
        ],
        out_specs=pl.BlockSpec((BM, D), lambda i: (i, 0)),
        out_shape=jax.ShapeDtypeStruct((N_NODES, D), jnp.float32),
    )(partials, W, b.reshape(1, D))


def kernel(x, edge_index, W, b):
    nw = 32
    nsuper = N_EDGES // (nw * G * K)
    ei = edge_index.astype(jnp.int32).reshape(2, nw, nsuper, G, K)
    partials = _sc_segment_partials(x, ei)
    return _tc_apply(partials, W, b)


# speedup vs baseline: 1.0117x; 1.0117x over previous
import functools

import jax
import jax.numpy as jnp
from jax import lax
from jax.experimental import pallas as pl
from jax.experimental.pallas import tpu as pltpu
from jax.experimental.pallas import tpu_sc as plsc

N_NODES = 10000
N_PAD = 10240
N_EDGES = 320000
D = 128
K = 125
G = 40
NB = 2
ZR = 32


def _sc_segment_partials(x, ei5d):
    info = plsc.get_sparse_core_info()
    nc, ns = info.num_cores, info.num_subcores
    nw = nc * ns
    e_per_w = N_EDGES // nw
    chunks = e_per_w // K
    nsuper = chunks // G
    rows_per_tile = N_PAD // ns

    mesh = plsc.VectorSubcoreMesh(core_axis_name="c", subcore_axis_name="s")

    @functools.partial(
        pl.kernel,
        out_type=jax.ShapeDtypeStruct((nc, N_PAD, D), jnp.float32),
        mesh=mesh,
        scratch_types=[
            pltpu.VMEM((G, K), jnp.int32),
            pltpu.VMEM((G, K), jnp.int32),
            [pltpu.VMEM((K, D), jnp.float32)] * NB,
            pltpu.VMEM((ZR, D), jnp.float32),
            pltpu.VMEM_SHARED((N_PAD, D), jnp.float32),
            [pltpu.SemaphoreType.DMA] * NB,
            [pltpu.SemaphoreType.DMA] * NB,
        ],
    )
    def seg_kernel(x_hbm, ei_hbm, out_hbm,
                   sidx, didx, rb, zbuf, acc, gsem, ssem):
        c = lax.axis_index("c")
        s = lax.axis_index("s")
        wid = s * nc + c

        pltpu.sync_copy(ei_hbm.at[0, wid, 0], sidx)
        pltpu.sync_copy(ei_hbm.at[1, wid, 0], didx)
        for b in range(NB):
            pltpu.async_copy(x_hbm.at[sidx.at[b]], rb[b], gsem[b])

        for r in range(ZR):
            for j in range(D // 16):
                zbuf[r, pl.ds(16 * j, 16)] = jnp.zeros((16,), jnp.float32)
        row_base = s * rows_per_tile

        @pl.loop(0, rows_per_tile // ZR)
        def _zero(k):
            pltpu.sync_copy(zbuf, acc.at[pl.ds(row_base + ZR * k, ZR)])

        plsc.subcore_barrier()

        for si in range(nsuper):
            if si > 0:
                pltpu.sync_copy(ei_hbm.at[0, wid, si], sidx)
                pltpu.sync_copy(ei_hbm.at[1, wid, si], didx)
                for b in range(NB):
                    pltpu.async_copy(x_hbm.at[sidx.at[b]], rb[b], gsem[b])

            @pl.loop(0, G, step=NB)
            def _edges(g):
                for b in range(NB):
                    pltpu.make_async_copy(
                        x_hbm.at[sidx.at[g + b]], rb[b], gsem[b]).wait()
                    pltpu.sync_copy(rb[b], acc.at[didx.at[g + b]], add=True)

                    @pl.when(g + NB + b < G)
                    def _(b=b):
                        pltpu.async_copy(
                            x_hbm.at[sidx.at[g + NB + b]], rb[b], gsem[b])

        plsc.subcore_barrier()
        pltpu.sync_copy(acc.at[pl.ds(row_base, rows_per_tile)],
                        out_hbm.at[c, pl.ds(row_base, rows_per_tile)])

    return seg_kernel(x, ei5d)


def _tc_apply(partials, W, b):
    nc = partials.shape[0]
    BM = 5000

    def mm(p_ref, w_ref, b_ref, o_ref):
        agg = p_ref[0]
        for i in range(1, nc):
            agg = agg + p_ref[i]
        o_ref[...] = (
            jnp.dot(agg, w_ref[...], preferred_element_type=jnp.float32)
            + b_ref[...]
        )

    return pl.pallas_call(
        mm,
        grid=(N_NODES // BM,),
        in_specs=[
            pl.BlockSpec((nc, BM, D), lambda i: (0, i, 0)),
            pl.BlockSpec((D, D), lambda i: (0, 0)),
            pl.BlockSpec((1, D), lambda i: (0, 0)),
        ],
        out_specs=pl.BlockSpec((BM, D), lambda i: (i, 0)),
        out_shape=jax.ShapeDtypeStruct((N_NODES, D), jnp.float32),
    )(partials, W, b.reshape(1, D))


def kernel(x, edge_index, W, b):
    nw = 32
    nsuper = N_EDGES // (nw * G * K)
    ei = edge_index.astype(jnp.int32).reshape(2, nw, nsuper, G, K)
    partials = _sc_segment_partials(x, ei)
    return _tc_apply(partials, W, b)
